# baseline (speedup 1.0000x reference)
import functools

import jax
import jax.numpy as jnp
from jax import lax
from jax.experimental import pallas as pl
from jax.experimental.pallas import tpu as pltpu
from jax.experimental.pallas import tpu_sc as plsc

N_NODES = 100000
NUM_TYPES = 128
L = 16
NC, NS = 2, 16
NW = NC * NS
C = 400
NCHUNKS = N_NODES // C
MAXK = (NCHUNKS + NW - 1) // NW

_mesh = plsc.VectorSubcoreMesh(core_axis_name="c", subcore_axis_name="s")


@functools.partial(
    pl.kernel,
    mesh=_mesh,
    compiler_params=pltpu.CompilerParams(needs_layout_passes=False),
    out_type=jax.ShapeDtypeStruct((N_NODES, NUM_TYPES), jnp.float32),
    scratch_types=[
        pltpu.VMEM((C,), jnp.int32),
        pltpu.VMEM((C,), jnp.int32),
        pltpu.VMEM((C, NUM_TYPES), jnp.float32),
        pltpu.VMEM((C, NUM_TYPES), jnp.float32),
        pltpu.SemaphoreType.DMA,
        pltpu.SemaphoreType.DMA,
    ],
)
def _onehot_sc(idx_hbm, zeros_hbm, out_hbm, idx0, idx1, buf0, buf1, sem0, sem1):
    wid = lax.axis_index("s") * NC + lax.axis_index("c")
    idxs, bufs, sems = (idx0, idx1), (buf0, buf1), (sem0, sem1)

    lane = lax.iota(jnp.int32, L)
    ones = jnp.full((L,), 1.0, jnp.float32)
    zeros = jnp.full((L,), 0.0, jnp.float32)

    def scatter(buf, idx_v, value):
        for g in range(C // L):
            iv = idx_v[pl.ds(g * L, L)]
            plsc.store_scatter(buf, [g * L + lane, iv], value)

    pltpu.async_copy(zeros_hbm, buf0, sem0)
    pltpu.async_copy(zeros_hbm, buf1, sem1)

    for k in range(MAXK):
        b = k % 2
        c = wid + NW * k

        @pl.when(c < NCHUNKS)
        def _(k=k, b=b, c=c):
            if k < 2:
                pltpu.make_async_copy(zeros_hbm, bufs[b], sems[b]).wait()
            else:
                pltpu.make_async_copy(
                    bufs[b], out_hbm.at[pl.ds((c - 2 * NW) * C, C)], sems[b]
                ).wait()
                scatter(bufs[b], idxs[b], zeros)
            pltpu.sync_copy(idx_hbm.at[pl.ds(c * C, C)], idxs[b])
            scatter(bufs[b], idxs[b], ones)
            pltpu.async_copy(bufs[b], out_hbm.at[pl.ds(c * C, C)], sems[b])

    pltpu.make_async_copy(buf0, out_hbm.at[pl.ds(0, C)], sem0).wait()
    pltpu.make_async_copy(buf1, out_hbm.at[pl.ds(0, C)], sem1).wait()


TC_BLOCK = 1000


def _onehot_tc_body(idx_ref, out_ref):
    iota = lax.broadcasted_iota(jnp.int32, (TC_BLOCK, NUM_TYPES), 1)
    out_ref[...] = (idx_ref[...] == iota).astype(jnp.float32)


def _onehot_tc(idx2d):
    return pl.pallas_call(
        _onehot_tc_body,
        grid=(N_NODES // TC_BLOCK,),
        in_specs=[pl.BlockSpec((TC_BLOCK, 1), lambda i: (i, 0))],
        out_specs=pl.BlockSpec((TC_BLOCK, NUM_TYPES), lambda i: (i, 0)),
        out_shape=jax.ShapeDtypeStruct((N_NODES, NUM_TYPES), jnp.float32),
    )(idx2d)


def kernel(atom_type, pos):
    idx2d = atom_type.astype(jnp.int32)
    idx = idx2d.reshape(-1)
    zeros_init = jnp.zeros((C, NUM_TYPES), jnp.float32)
    node_attrs = _onehot_sc(idx, zeros_init)
    node_features = _onehot_tc(idx2d)
    return (node_attrs, node_features)


# speedup vs baseline: 1.5043x; 1.5043x over previous
import functools

import jax
import jax.numpy as jnp
from jax import lax
from jax.experimental import pallas as pl
from jax.experimental.pallas import tpu as pltpu
from jax.experimental.pallas import tpu_sc as plsc

N_NODES = 100000
NUM_TYPES = 128
L = 16
NC, NS = 2, 16
NW = NC * NS
C = 400
NCHUNKS = N_NODES // C
MAXK = (NCHUNKS + NW - 1) // NW

_mesh = plsc.VectorSubcoreMesh(core_axis_name="c", subcore_axis_name="s")


@functools.partial(
    pl.kernel,
    mesh=_mesh,
    compiler_params=pltpu.CompilerParams(needs_layout_passes=False),
    out_type=jax.ShapeDtypeStruct((N_NODES, NUM_TYPES), jnp.float32),
    scratch_types=[
        pltpu.VMEM((C,), jnp.int32),
        pltpu.VMEM((C,), jnp.int32),
        pltpu.VMEM((C, NUM_TYPES), jnp.float32),
        pltpu.VMEM((C, NUM_TYPES), jnp.float32),
        pltpu.SemaphoreType.DMA,
        pltpu.SemaphoreType.DMA,
    ],
)
def _onehot_sc(idx_hbm, zeros_hbm, out_hbm, idx0, idx1, buf0, buf1, sem0, sem1):
    wid = lax.axis_index("s") * NC + lax.axis_index("c")
    idxs, bufs, sems = (idx0, idx1), (buf0, buf1), (sem0, sem1)

    lane = lax.iota(jnp.int32, L)
    ones = jnp.full((L,), 1.0, jnp.float32)
    zeros = jnp.full((L,), 0.0, jnp.float32)

    def scatter(buf, idx_v, value):
        for g in range(C // L):
            iv = idx_v[pl.ds(g * L, L)]
            plsc.store_scatter(buf, [g * L + lane, iv], value)

    pltpu.async_copy(zeros_hbm, buf0, sem0)
    pltpu.async_copy(zeros_hbm, buf1, sem1)

    for k in range(MAXK):
        b = k % 2
        c = wid + NW * k

        @pl.when(c < NCHUNKS)
        def _(k=k, b=b, c=c):
            if k < 2:
                pltpu.make_async_copy(zeros_hbm, bufs[b], sems[b]).wait()
            else:
                pltpu.make_async_copy(
                    bufs[b], out_hbm.at[pl.ds((c - 2 * NW) * C, C)], sems[b]
                ).wait()
                scatter(bufs[b], idxs[b], zeros)
            pltpu.sync_copy(idx_hbm.at[pl.ds(c * C, C)], idxs[b])
            scatter(bufs[b], idxs[b], ones)
            pltpu.async_copy(bufs[b], out_hbm.at[pl.ds(c * C, C)], sems[b])

    pltpu.make_async_copy(buf0, out_hbm.at[pl.ds(0, C)], sem0).wait()
    pltpu.make_async_copy(buf1, out_hbm.at[pl.ds(0, C)], sem1).wait()


TC_BLOCK = 1024
IDX_ROWS = 784


def _onehot_tc_body(idx_ref, out_ref):
    idx_tile = idx_ref[...]
    cls = lax.broadcasted_iota(jnp.int32, (NUM_TYPES, 128), 0)
    for s in range(8):
        row = idx_tile[s:s + 1, :]
        pt = (cls == row).astype(jnp.float32)
        out_ref[pl.ds(s * 128, 128), :] = pt.T


def _onehot_tc(idx_pad):
    return pl.pallas_call(
        _onehot_tc_body,
        grid=(pl.cdiv(N_NODES, TC_BLOCK),),
        in_specs=[pl.BlockSpec((8, 128), lambda i: (i, 0))],
        out_specs=pl.BlockSpec((TC_BLOCK, NUM_TYPES), lambda i: (i, 0)),
        out_shape=jax.ShapeDtypeStruct((N_NODES, NUM_TYPES), jnp.float32),
    )(idx_pad)


def kernel(atom_type, pos):
    idx = atom_type.reshape(-1).astype(jnp.int32)
    idx_pad = jnp.zeros((IDX_ROWS * 128,), jnp.int32).at[:N_NODES].set(idx)
    zeros_init = jnp.zeros((C, NUM_TYPES), jnp.float32)
    node_attrs = _onehot_sc(idx, zeros_init)
    node_features = _onehot_tc(idx_pad.reshape(IDX_ROWS, 128))
    return (node_attrs, node_features)


# speedup vs baseline: 1.8930x; 1.2584x over previous
import functools

import jax
import jax.numpy as jnp
from jax import lax
from jax.experimental import pallas as pl
from jax.experimental.pallas import tpu as pltpu
from jax.experimental.pallas import tpu_sc as plsc

N_NODES = 100000
NUM_TYPES = 128
L = 16
NC, NS = 2, 16
NW = NC * NS
C = 400
NCHUNKS = N_NODES // C
MAXK = (NCHUNKS + NW - 1) // NW

_mesh = plsc.VectorSubcoreMesh(core_axis_name="c", subcore_axis_name="s")


@functools.partial(
    pl.kernel,
    mesh=_mesh,
    compiler_params=pltpu.CompilerParams(needs_layout_passes=False),
    out_type=jax.ShapeDtypeStruct((N_NODES, NUM_TYPES), jnp.float32),
    scratch_types=[
        pltpu.VMEM((C,), jnp.int32),
        pltpu.VMEM((C,), jnp.int32),
        pltpu.VMEM((C, NUM_TYPES), jnp.float32),
        pltpu.VMEM((C, NUM_TYPES), jnp.float32),
        pltpu.SemaphoreType.DMA,
        pltpu.SemaphoreType.DMA,
    ],
)
def _onehot_sc(idx_hbm, zeros_hbm, out_hbm, idx0, idx1, buf0, buf1, sem0, sem1):
    wid = lax.axis_index("s") * NC + lax.axis_index("c")
    idxs, bufs, sems = (idx0, idx1), (buf0, buf1), (sem0, sem1)

    lane = lax.iota(jnp.int32, L)
    ones = jnp.full((L,), 1.0, jnp.float32)
    zeros = jnp.full((L,), 0.0, jnp.float32)

    def scatter(buf, idx_v, value):
        for g in range(C // L):
            iv = idx_v[pl.ds(g * L, L)]
            plsc.store_scatter(buf, [g * L + lane, iv], value)

    pltpu.async_copy(zeros_hbm, buf0, sem0)
    pltpu.async_copy(zeros_hbm, buf1, sem1)

    for k in range(MAXK):
        b = k % 2
        c = wid + NW * k

        @pl.when(c < NCHUNKS)
        def _(k=k, b=b, c=c):
            if k < 2:
                pltpu.make_async_copy(zeros_hbm, bufs[b], sems[b]).wait()
            else:
                pltpu.make_async_copy(
                    bufs[b], out_hbm.at[pl.ds((c - 2 * NW) * C, C)], sems[b]
                ).wait()
                scatter(bufs[b], idxs[b], zeros)
            pltpu.sync_copy(idx_hbm.at[pl.ds(c * C, C)], idxs[b])
            scatter(bufs[b], idxs[b], ones)
            pltpu.async_copy(bufs[b], out_hbm.at[pl.ds(c * C, C)], sems[b])

    pltpu.make_async_copy(buf0, out_hbm.at[pl.ds(0, C)], sem0).wait()
    pltpu.make_async_copy(buf1, out_hbm.at[pl.ds(0, C)], sem1).wait()


TC_BLOCK = 4096
IDX_ROWS = 800


def _onehot_tc_body(idx_ref, out_ref):
    idx_tile = idx_ref[...]
    cls = lax.broadcasted_iota(jnp.int32, (NUM_TYPES, 128), 0)
    for s in range(TC_BLOCK // 128):
        row = idx_tile[s:s + 1, :]
        pt = (cls == row).astype(jnp.float32)
        out_ref[pl.ds(s * 128, 128), :] = pt.T


def _onehot_tc(idx_pad):
    return pl.pallas_call(
        _onehot_tc_body,
        grid=(pl.cdiv(N_NODES, TC_BLOCK),),
        in_specs=[pl.BlockSpec((TC_BLOCK // 128, 128), lambda i: (i, 0))],
        out_specs=pl.BlockSpec((TC_BLOCK, NUM_TYPES), lambda i: (i, 0)),
        out_shape=jax.ShapeDtypeStruct((N_NODES, NUM_TYPES), jnp.float32),
    )(idx_pad)


def kernel(atom_type, pos):
    idx = atom_type.reshape(-1).astype(jnp.int32)
    idx_pad = jnp.zeros((IDX_ROWS * 128,), jnp.int32).at[:N_NODES].set(idx)
    zeros_init = jnp.zeros((C, NUM_TYPES), jnp.float32)
    node_attrs = _onehot_sc(idx, zeros_init)
    node_features = _onehot_tc(idx_pad.reshape(IDX_ROWS, 128))
    return (node_attrs, node_features)


# speedup vs baseline: 1.8957x; 1.0014x over previous
import functools

import jax
import jax.numpy as jnp
from jax import lax
from jax.experimental import pallas as pl
from jax.experimental.pallas import tpu as pltpu
from jax.experimental.pallas import tpu_sc as plsc

N_NODES = 100000
NUM_TYPES = 128
L = 16
NC, NS = 2, 16
NW = NC * NS
C = 400
NCHUNKS = N_NODES // C
MAXK = (NCHUNKS + NW - 1) // NW

_mesh = plsc.VectorSubcoreMesh(core_axis_name="c", subcore_axis_name="s")


@functools.partial(
    pl.kernel,
    mesh=_mesh,
    compiler_params=pltpu.CompilerParams(needs_layout_passes=False),
    out_type=jax.ShapeDtypeStruct((N_NODES, NUM_TYPES), jnp.float32),
    scratch_types=[
        pltpu.VMEM((C,), jnp.int32),
        pltpu.VMEM((C,), jnp.int32),
        pltpu.VMEM((C, NUM_TYPES), jnp.float32),
        pltpu.VMEM((C, NUM_TYPES), jnp.float32),
        pltpu.SemaphoreType.DMA,
        pltpu.SemaphoreType.DMA,
    ],
)
def _onehot_sc(idx_hbm, zeros_hbm, out_hbm, idx0, idx1, buf0, buf1, sem0, sem1):
    wid = lax.axis_index("s") * NC + lax.axis_index("c")
    idxs, bufs, sems = (idx0, idx1), (buf0, buf1), (sem0, sem1)

    lane = lax.iota(jnp.int32, L)
    ones = jnp.full((L,), 1.0, jnp.float32)
    zeros = jnp.full((L,), 0.0, jnp.float32)

    def scatter(buf, idx_v, value):
        for g in range(C // L):
            iv = idx_v[pl.ds(g * L, L)]
            plsc.store_scatter(buf, [g * L + lane, iv], value)

    pltpu.async_copy(zeros_hbm, buf0, sem0)
    pltpu.async_copy(zeros_hbm, buf1, sem1)

    for k in range(MAXK):
        b = k % 2
        c = wid + NW * k

        @pl.when(c < NCHUNKS)
        def _(k=k, b=b, c=c):
            if k < 2:
                pltpu.make_async_copy(zeros_hbm, bufs[b], sems[b]).wait()
            else:
                pltpu.make_async_copy(
                    bufs[b], out_hbm.at[pl.ds((c - 2 * NW) * C, C)], sems[b]
                ).wait()
                scatter(bufs[b], idxs[b], zeros)
            pltpu.sync_copy(idx_hbm.at[pl.ds(c * C, C)], idxs[b])
            scatter(bufs[b], idxs[b], ones)
            pltpu.async_copy(bufs[b], out_hbm.at[pl.ds(c * C, C)], sems[b])

    pltpu.make_async_copy(buf0, out_hbm.at[pl.ds(0, C)], sem0).wait()
    pltpu.make_async_copy(buf1, out_hbm.at[pl.ds(0, C)], sem1).wait()


TC_BLOCK = 8192
IDX_ROWS = 832


def _onehot_tc_body(idx_ref, out_ref):
    idx_tile = idx_ref[...]
    cls = lax.broadcasted_iota(jnp.int32, (NUM_TYPES, 128), 0)
    for s in range(TC_BLOCK // 128):
        row = idx_tile[s:s + 1, :]
        pt = (cls == row).astype(jnp.float32)
        out_ref[pl.ds(s * 128, 128), :] = pt.T


def _onehot_tc(idx_pad):
    return pl.pallas_call(
        _onehot_tc_body,
        grid=(pl.cdiv(N_NODES, TC_BLOCK),),
        in_specs=[pl.BlockSpec((TC_BLOCK // 128, 128), lambda i: (i, 0))],
        out_specs=pl.BlockSpec((TC_BLOCK, NUM_TYPES), lambda i: (i, 0)),
        out_shape=jax.ShapeDtypeStruct((N_NODES, NUM_TYPES), jnp.float32),
    )(idx_pad)


def kernel(atom_type, pos):
    idx = atom_type.reshape(-1).astype(jnp.int32)
    idx_pad = jnp.zeros((IDX_ROWS * 128,), jnp.int32).at[:N_NODES].set(idx)
    zeros_init = jnp.zeros((C, NUM_TYPES), jnp.float32)
    node_attrs = _onehot_sc(idx, zeros_init)
    node_features = _onehot_tc(idx_pad.reshape(IDX_ROWS, 128))
    return (node_attrs, node_features)


# speedup vs baseline: 2.4943x; 1.3158x over previous
import functools

import jax
import jax.numpy as jnp
from jax import lax
from jax.experimental import pallas as pl
from jax.experimental.pallas import tpu as pltpu
from jax.experimental.pallas import tpu_sc as plsc

N_NODES = 100000
NUM_TYPES = 128
L = 16
NC, NS = 2, 16
NW = NC * NS
C = 400
NCHUNKS = N_NODES // C
MAXK = (NCHUNKS + NW - 1) // NW

_mesh = plsc.VectorSubcoreMesh(core_axis_name="c", subcore_axis_name="s")


@functools.partial(
    pl.kernel,
    mesh=_mesh,
    compiler_params=pltpu.CompilerParams(needs_layout_passes=False),
    out_type=jax.ShapeDtypeStruct((N_NODES, NUM_TYPES), jnp.float32),
    scratch_types=[
        pltpu.VMEM((C,), jnp.int32),
        pltpu.VMEM((C,), jnp.int32),
        pltpu.VMEM((C, NUM_TYPES), jnp.float32),
        pltpu.VMEM((C, NUM_TYPES), jnp.float32),
        pltpu.VMEM_SHARED((C, NUM_TYPES), jnp.float32),
        pltpu.SemaphoreType.DMA,
        pltpu.SemaphoreType.DMA,
    ],
)
def _onehot_sc(idx_hbm, zeros_hbm, out_hbm, idx0, idx1, buf0, buf1, zshared,
               sem0, sem1):
    wid = lax.axis_index("s") * NC + lax.axis_index("c")
    idxs, bufs, sems = (idx0, idx1), (buf0, buf1), (sem0, sem1)

    lane = lax.iota(jnp.int32, L)
    ones = jnp.full((L,), 1.0, jnp.float32)
    zeros = jnp.full((L,), 0.0, jnp.float32)

    def scatter(buf, idx_v, value):
        for g in range(C // L):
            iv = idx_v[pl.ds(g * L, L)]
            plsc.store_scatter(buf, [g * L + lane, iv], value)

    @pl.when(lax.axis_index("s") == 0)
    def _():
        pltpu.sync_copy(zeros_hbm, zshared)

    plsc.subcore_barrier()
    pltpu.async_copy(zshared, buf0, sem0)
    pltpu.async_copy(zshared, buf1, sem1)

    for k in range(MAXK):
        b = k % 2
        c = wid + NW * k

        @pl.when(c < NCHUNKS)
        def _(k=k, b=b, c=c):
            if k < 2:
                pltpu.make_async_copy(zshared, bufs[b], sems[b]).wait()
            else:
                pltpu.make_async_copy(
                    bufs[b], out_hbm.at[pl.ds((c - 2 * NW) * C, C)], sems[b]
                ).wait()
                scatter(bufs[b], idxs[b], zeros)
            pltpu.sync_copy(idx_hbm.at[pl.ds(c * C, C)], idxs[b])
            scatter(bufs[b], idxs[b], ones)
            pltpu.async_copy(bufs[b], out_hbm.at[pl.ds(c * C, C)], sems[b])

    pltpu.make_async_copy(buf0, out_hbm.at[pl.ds(0, C)], sem0).wait()
    pltpu.make_async_copy(buf1, out_hbm.at[pl.ds(0, C)], sem1).wait()


TC_BLOCK = 8192
IDX_ROWS = 832


def _onehot_tc_body(idx_ref, out_ref):
    idx_tile = idx_ref[...]
    cls = lax.broadcasted_iota(jnp.int32, (NUM_TYPES, 128), 0)
    for s in range(TC_BLOCK // 128):
        row = idx_tile[s:s + 1, :]
        pt = (cls == row).astype(jnp.float32)
        out_ref[pl.ds(s * 128, 128), :] = pt.T


def _onehot_tc(idx_pad):
    return pl.pallas_call(
        _onehot_tc_body,
        grid=(pl.cdiv(N_NODES, TC_BLOCK),),
        in_specs=[pl.BlockSpec((TC_BLOCK // 128, 128), lambda i: (i, 0))],
        out_specs=pl.BlockSpec((TC_BLOCK, NUM_TYPES), lambda i: (i, 0)),
        out_shape=jax.ShapeDtypeStruct((N_NODES, NUM_TYPES), jnp.float32),
    )(idx_pad)


def kernel(atom_type, pos):
    idx = atom_type.reshape(-1).astype(jnp.int32)
    idx_pad = jnp.zeros((IDX_ROWS * 128,), jnp.int32).at[:N_NODES].set(idx)
    zeros_init = jnp.zeros((C, NUM_TYPES), jnp.float32)
    node_attrs = _onehot_sc(idx, zeros_init)
    node_features = _onehot_tc(idx_pad.reshape(IDX_ROWS, 128))
    return (node_attrs, node_features)


# speedup vs baseline: 2.5183x; 1.0096x over previous
import functools

import jax
import jax.numpy as jnp
from jax import lax
from jax.experimental import pallas as pl
from jax.experimental.pallas import tpu as pltpu
from jax.experimental.pallas import tpu_sc as plsc

N_NODES = 100000
NUM_TYPES = 128
L = 16
NC, NS = 2, 16
NW = NC * NS
C = 400
NCHUNKS = N_NODES // C
MAXK = (NCHUNKS + NW - 1) // NW

_mesh = plsc.VectorSubcoreMesh(core_axis_name="c", subcore_axis_name="s")


@functools.partial(
    pl.kernel,
    mesh=_mesh,
    compiler_params=pltpu.CompilerParams(needs_layout_passes=False),
    out_type=jax.ShapeDtypeStruct((N_NODES, NUM_TYPES), jnp.float32),
    scratch_types=[
        *[pltpu.VMEM((C,), jnp.int32) for _ in range(MAXK)],
        pltpu.VMEM((C, NUM_TYPES), jnp.float32),
        pltpu.VMEM((C, NUM_TYPES), jnp.float32),
        pltpu.VMEM_SHARED((C, NUM_TYPES), jnp.float32),
        pltpu.SemaphoreType.DMA,
        pltpu.SemaphoreType.DMA,
        pltpu.SemaphoreType.DMA,
    ],
)
def _onehot_sc(idx_hbm, zeros_hbm, out_hbm, *refs):
    idxv = refs[:MAXK]
    buf0, buf1, zshared, sem0, sem1, sem_idx = refs[MAXK:]
    wid = lax.axis_index("s") * NC + lax.axis_index("c")
    bufs, sems = (buf0, buf1), (sem0, sem1)

    lane = lax.iota(jnp.int32, L)
    ones = jnp.full((L,), 1.0, jnp.float32)
    zeros = jnp.full((L,), 0.0, jnp.float32)

    def scatter(buf, k, value):
        for g in range(C // L):
            iv = idxv[k][pl.ds(g * L, L)]
            plsc.store_scatter(buf, [g * L + lane, iv], value)

    for k in range(MAXK):
        c = wid + NW * k

        @pl.when(c < NCHUNKS)
        def _(k=k, c=c):
            pltpu.async_copy(idx_hbm.at[pl.ds(c * C, C)], idxv[k], sem_idx)

    @pl.when(lax.axis_index("s") == 0)
    def _():
        pltpu.sync_copy(zeros_hbm, zshared)

    plsc.subcore_barrier()
    pltpu.async_copy(zshared, buf0, sem0)
    pltpu.async_copy(zshared, buf1, sem1)

    for k in range(MAXK):
        c = wid + NW * k

        @pl.when(c < NCHUNKS)
        def _(k=k, c=c):
            pltpu.make_async_copy(
                idx_hbm.at[pl.ds(c * C, C)], idxv[k], sem_idx
            ).wait()

    for k in range(MAXK):
        b = k % 2
        c = wid + NW * k

        @pl.when(c < NCHUNKS)
        def _(k=k, b=b, c=c):
            if k < 2:
                pltpu.make_async_copy(zshared, bufs[b], sems[b]).wait()
            else:
                pltpu.make_async_copy(
                    bufs[b], out_hbm.at[pl.ds((c - 2 * NW) * C, C)], sems[b]
                ).wait()
                scatter(bufs[b], k - 2, zeros)
            scatter(bufs[b], k, ones)
            pltpu.async_copy(bufs[b], out_hbm.at[pl.ds(c * C, C)], sems[b])

    pltpu.make_async_copy(buf0, out_hbm.at[pl.ds(0, C)], sem0).wait()
    pltpu.make_async_copy(buf1, out_hbm.at[pl.ds(0, C)], sem1).wait()


TC_BLOCK = 8192
IDX_ROWS = 832


def _onehot_tc_body(idx_ref, out_ref):
    idx_tile = idx_ref[...]
    cls = lax.broadcasted_iota(jnp.int32, (NUM_TYPES, 128), 0)
    for s in range(TC_BLOCK // 128):
        row = idx_tile[s:s + 1, :]
        pt = (cls == row).astype(jnp.float32)
        out_ref[pl.ds(s * 128, 128), :] = pt.T


def _onehot_tc(idx_pad):
    return pl.pallas_call(
        _onehot_tc_body,
        grid=(pl.cdiv(N_NODES, TC_BLOCK),),
        in_specs=[pl.BlockSpec((TC_BLOCK // 128, 128), lambda i: (i, 0))],
        out_specs=pl.BlockSpec((TC_BLOCK, NUM_TYPES), lambda i: (i, 0)),
        out_shape=jax.ShapeDtypeStruct((N_NODES, NUM_TYPES), jnp.float32),
    )(idx_pad)


def kernel(atom_type, pos):
    idx = atom_type.reshape(-1).astype(jnp.int32)
    idx_pad = jnp.zeros((IDX_ROWS * 128,), jnp.int32).at[:N_NODES].set(idx)
    zeros_init = jnp.zeros((C, NUM_TYPES), jnp.float32)
    node_attrs = _onehot_sc(idx, zeros_init)
    node_features = _onehot_tc(idx_pad.reshape(IDX_ROWS, 128))
    return (node_attrs, node_features)
